# baseline (speedup 1.0000x reference)
import math

import jax
import jax.numpy as jnp
import numpy as np
from jax.experimental import pallas as pl
from jax.experimental.pallas import tpu as pltpu

H = 12
HC = 6
DH = 4
DV = 16
SCALE = 1.0 / math.sqrt(DH)
LOG2E = 1.4426950408889634
BQ = 512


def _np_chamber_mats():
    c1 = np.zeros((24, 96), np.float32)
    o1 = np.zeros((24, 1), np.float32)
    for h in range(H):
        for j in range(2):
            c1[h * 2 + j, h * 8 + 0] = 2 * j - 1
            o1[h * 2 + j, 0] = 1 - j
    b2 = np.zeros((48, 24), np.float32)
    c2 = np.zeros((48, 96), np.float32)
    o2 = np.zeros((48, 1), np.float32)
    for h in range(H):
        for c in range(4):
            j = (c >> 1) & 1
            b2[h * 4 + c, h * 2 + (c & 1)] = 1.0
            c2[h * 4 + c, h * 8 + 1] = 2 * j - 1
            o2[h * 4 + c, 0] = 1 - j
    b3 = np.zeros((96, 48), np.float32)
    c3 = np.zeros((96, 96), np.float32)
    o3 = np.zeros((96, 1), np.float32)
    for h in range(H):
        for c in range(8):
            j = (c >> 2) & 1
            b3[h * 8 + c, h * 4 + (c & 3)] = 1.0
            c3[h * 8 + c, h * 8 + 2] = 2 * j - 1
            o3[h * 8 + c, 0] = 1 - j
    b4 = np.zeros((192, 96), np.float32)
    c4 = np.zeros((192, 96), np.float32)
    o4 = np.zeros((192, 1), np.float32)
    for h in range(H):
        for c in range(16):
            j = (c >> 3) & 1
            b4[h * 16 + c, h * 8 + (c & 7)] = 1.0
            c4[h * 16 + c, h * 8 + 3] = 2 * j - 1
            o4[h * 16 + c, 0] = 1 - j
    return c1, o1, b2, c2, o2, b3, c3, o3, b4, c4, o4


def _np_amask():
    r = np.arange(BQ)[:, None]
    c = np.arange(BQ)[None, :]
    return np.where(r > c, np.float32(-1e30), np.float32(0.0))


def _np_onec():
    o = np.zeros((96, 1), np.float32)
    for h in range(H):
        o[h * 8 + DH, 0] = LOG2E
    return o


_G96 = np.kron(np.eye(H, dtype=np.float32), np.ones((8, 8), np.float32))
_PAD = np.kron(np.eye(H, dtype=np.float32),
               np.eye(8, DH, dtype=np.float32))
_TILE4 = np.kron(np.ones((1, H), np.float32),
                 np.eye(DH, 8, dtype=np.float32))
_R8 = np.kron(np.ones((H, 1), np.float32),
              np.eye(8, DH, dtype=np.float32))
_SEL4 = np.kron(np.eye(H, dtype=np.float32),
                (np.arange(8) == DH).astype(np.float32)[:, None])
_TILE16 = np.kron(np.ones((1, H), np.float32),
                  np.eye(16, dtype=np.float32))
_BLK16 = np.kron(np.eye(H, dtype=np.float32),
                 np.ones((8, 16), np.float32))
_EXPD6 = np.kron(np.eye(HC, dtype=np.float32),
                 np.ones((DV, 1), np.float32))
_ONEC = _np_onec()
_CH = _np_chamber_mats()
_AMASK = _np_amask()


def _fused_kernel(x_ref, wq_ref, wk_ref, wv_ref, wn_ref, roots_ref, cb_ref,
                  pad_ref, tile4_ref, r8_ref, sel4_ref, tile16_ref,
                  blk16_ref, g_ref, onec_ref, c1_ref, o1_ref, b2_ref, c2_ref,
                  o2_ref, b3_ref, c3_ref, o3_ref, b4_ref, c4_ref, o4_ref,
                  amask_ref, exp_ref, wout_ref, y_ref,
                  qs_ref, ks_ref, vs_ref, ot_ref, l_ref):
    f32 = jnp.float32
    bf16 = jnp.bfloat16
    x = x_ref[...]
    g = g_ref[...]
    pad = pad_ref[...]
    tile4 = tile4_ref[...]
    t = x.shape[0]

    def dgt(a, b):
        return jax.lax.dot_general(a, b, (((1,), (1,)), ((), ())),
                                   preferred_element_type=f32)

    def dtg(a, b):
        return jax.lax.dot_general(a, b, (((0,), (0,)), ((), ())),
                                   preferred_element_type=f32)

    def mm(a, b):
        return jnp.dot(a, b, preferred_element_type=f32)

    def norm_t(a):
        n2 = mm(g, a * a)
        return a / jnp.maximum(jnp.sqrt(n2), 1e-12)

    qt = norm_t(dgt(mm(pad, wq_ref[...]), x))
    kt = norm_t(dgt(mm(pad, wk_ref[...]), x))
    vt = dgt(wv_ref[...], x)

    bdt = mm(mm(pad, wn_ref[...]), tile4) * g
    qt = norm_t(dtg(bdt, qt)) * (SCALE * LOG2E) + onec_ref[...]

    bdr = mm(mm(r8_ref[...], roots_ref[...]), tile4) * g
    ssg = jax.nn.sigmoid(mm(bdr, kt) * 3.0)
    u = mm(c1_ref[...], ssg) + o1_ref[...]
    u = mm(b2_ref[...], u) * (mm(c2_ref[...], ssg) + o2_ref[...])
    u = mm(b3_ref[...], u) * (mm(c3_ref[...], ssg) + o3_ref[...])
    u = mm(b4_ref[...], u) * (mm(c4_ref[...], ssg) + o4_ref[...])
    cbe = mm(sel4_ref[...], cb_ref[...])
    cbe = mm(cbe, tile16_ref[...]) * blk16_ref[...]
    kt = kt + mm(cbe, u)

    qs_ref[...] = qt.astype(bf16)
    ks_ref[...] = kt.astype(bf16)
    vs_ref[...] = vt.astype(bf16)
    amask = amask_ref[...]
    ones_row = jnp.ones((1, t), bf16)
    i = pl.program_id(0)

    def sdot(ks, qs):
        s = jax.lax.dot_general(ks, qs, (((0,), (0,)), ((), ())),
                                preferred_element_type=f32)
        return s.astype(bf16)

    for j in range(HC):
        base = i * (HC * 8) + j * 8
        vb = i * (HC * DV) + j * DV
        qh = qs_ref[pl.ds(base, 8), :]
        kh = ks_ref[pl.ds(base, 8), :]
        vh = jnp.concatenate([vs_ref[pl.ds(vb, DV), :], ones_row], axis=0)
        for qb in range(t // BQ):
            lo = qb * BQ
            qblk = qh[:, lo:lo + BQ]
            p1 = jnp.exp2(sdot(kh[:, lo:lo + BQ], qblk) + amask)
            oa = mm(vh[:, lo:lo + BQ], p1)
            if qb:
                p0 = jnp.exp2(sdot(kh[:, :lo], qblk))
                oa = oa + mm(vh[:, :lo], p0)
            ot_ref[j * DV:(j + 1) * DV, lo:lo + BQ] = oa[:DV, :]
            l_ref[j:j + 1, lo:lo + BQ] = oa[DV:DV + 1, :]
    lbig = mm(exp_ref[...], 1.0 / l_ref[...])
    y_ref[0] = jax.lax.dot_general(ot_ref[...] * lbig, wout_ref[0],
                                   (((0,), (1,)), ((), ())),
                                   preferred_element_type=f32)


def _full(shape):
    nd = len(shape)
    return pl.BlockSpec(shape, lambda i, _n=nd: (0,) * _n)


def kernel(x, Wq, Wk, Wv, Wout, W_nudge, chamber_bonus, simple_roots):
    b, t, d_model = x.shape
    c1, o1, b2, c2, o2, b3, c3, o3, b4, c4, o4 = (jnp.asarray(m)
                                                  for m in _CH)
    operands = (
        x.reshape(t, d_model), Wq, Wk, Wv, W_nudge.reshape(H * DH, DH),
        simple_roots, chamber_bonus, jnp.asarray(_PAD), jnp.asarray(_TILE4),
        jnp.asarray(_R8), jnp.asarray(_SEL4), jnp.asarray(_TILE16),
        jnp.asarray(_BLK16), jnp.asarray(_G96), jnp.asarray(_ONEC), c1, o1,
        b2, c2, o2, b3, c3, o3, b4, c4, o4,
        jnp.asarray(_AMASK, dtype=jnp.bfloat16), jnp.asarray(_EXPD6),
        Wout.reshape(d_model, 2, HC * DV).transpose(1, 0, 2))
    in_specs = [_full(op.shape) for op in operands[:-1]]
    in_specs.append(pl.BlockSpec((1, d_model, HC * DV), lambda i: (i, 0, 0)))
    y2 = pl.pallas_call(
        _fused_kernel,
        grid=(2,),
        in_specs=in_specs,
        out_specs=pl.BlockSpec((1, t, d_model), lambda i: (i, 0, 0)),
        out_shape=jax.ShapeDtypeStruct((2, t, d_model), jnp.float32),
        scratch_shapes=[pltpu.VMEM((96, t), jnp.bfloat16),
                        pltpu.VMEM((96, t), jnp.bfloat16),
                        pltpu.VMEM((H * DV, t), jnp.bfloat16),
                        pltpu.VMEM((HC * DV, t), jnp.float32),
                        pltpu.VMEM((HC, t), jnp.float32)],
        compiler_params=pltpu.CompilerParams(
            dimension_semantics=("parallel",)),
    )(*operands)
    return (y2[0] + y2[1]).reshape(b, t, d_model)


# speedup vs baseline: 1.3077x; 1.3077x over previous
import math

import jax
import jax.numpy as jnp
import numpy as np
from jax.experimental import pallas as pl
from jax.experimental.pallas import tpu as pltpu

H = 12
DH = 4
DV = 16
SCALE = 1.0 / math.sqrt(DH)
LOG2E = 1.4426950408889634
BQ = 512


def _np_chamber_mats():
    c1 = np.zeros((24, 96), np.float32)
    o1 = np.zeros((24, 1), np.float32)
    for h in range(H):
        for j in range(2):
            c1[h * 2 + j, h * 8 + 0] = 2 * j - 1
            o1[h * 2 + j, 0] = 1 - j
    b2 = np.zeros((48, 24), np.float32)
    c2 = np.zeros((48, 96), np.float32)
    o2 = np.zeros((48, 1), np.float32)
    for h in range(H):
        for c in range(4):
            j = (c >> 1) & 1
            b2[h * 4 + c, h * 2 + (c & 1)] = 1.0
            c2[h * 4 + c, h * 8 + 1] = 2 * j - 1
            o2[h * 4 + c, 0] = 1 - j
    b3 = np.zeros((96, 48), np.float32)
    c3 = np.zeros((96, 96), np.float32)
    o3 = np.zeros((96, 1), np.float32)
    for h in range(H):
        for c in range(8):
            j = (c >> 2) & 1
            b3[h * 8 + c, h * 4 + (c & 3)] = 1.0
            c3[h * 8 + c, h * 8 + 2] = 2 * j - 1
            o3[h * 8 + c, 0] = 1 - j
    b4 = np.zeros((192, 96), np.float32)
    c4 = np.zeros((192, 96), np.float32)
    o4 = np.zeros((192, 1), np.float32)
    for h in range(H):
        for c in range(16):
            j = (c >> 3) & 1
            b4[h * 16 + c, h * 8 + (c & 7)] = 1.0
            c4[h * 16 + c, h * 8 + 3] = 2 * j - 1
            o4[h * 16 + c, 0] = 1 - j
    return c1, o1, b2, c2, o2, b3, c3, o3, b4, c4, o4


def _np_amask():
    r = np.arange(BQ)[:, None]
    c = np.arange(BQ)[None, :]
    return np.where(r > c, np.float32(-1e30), np.float32(0.0))


def _np_onec():
    o = np.zeros((96, 1), np.float32)
    for h in range(H):
        o[h * 8 + DH, 0] = LOG2E
    return o


_G96 = np.kron(np.eye(H, dtype=np.float32), np.ones((8, 8), np.float32))
_PAD = np.kron(np.eye(H, dtype=np.float32),
               np.eye(8, DH, dtype=np.float32))
_TILE4 = np.kron(np.ones((1, H), np.float32),
                 np.eye(DH, 8, dtype=np.float32))
_R8 = np.kron(np.ones((H, 1), np.float32),
              np.eye(8, DH, dtype=np.float32))
_SEL4 = np.kron(np.eye(H, dtype=np.float32),
                (np.arange(8) == DH).astype(np.float32)[:, None])
_TILE16 = np.kron(np.ones((1, H), np.float32),
                  np.eye(16, dtype=np.float32))
_BLK16 = np.kron(np.eye(H, dtype=np.float32),
                 np.ones((8, 16), np.float32))
_EXPD = np.kron(np.eye(H, dtype=np.float32),
                np.ones((DV, 1), np.float32))
_ONEC = _np_onec()
_CH = _np_chamber_mats()
_AMASK4 = np.tile(_np_amask(), (1, 4))
_QMASK = np.kron(np.eye(4, dtype=np.float32),
                 np.ones((8, BQ), np.float32))


def _fused_kernel(x_ref, wq_ref, wk_ref, wv_ref, wn_ref, roots_ref, cb_ref,
                  pad_ref, tile4_ref, r8_ref, sel4_ref, tile16_ref,
                  blk16_ref, g_ref, onec_ref, c1_ref, o1_ref, b2_ref, c2_ref,
                  o2_ref, b3_ref, c3_ref, o3_ref, b4_ref, c4_ref, o4_ref,
                  amask_ref, qmask_ref, exp_ref, wout_ref, y_ref, ot_ref,
                  l_ref):
    f32 = jnp.float32
    bf16 = jnp.bfloat16
    x = x_ref[...]
    g = g_ref[...]
    pad = pad_ref[...]
    tile4 = tile4_ref[...]
    t = x.shape[0]

    def dgt(a, b):
        return jax.lax.dot_general(a, b, (((1,), (1,)), ((), ())),
                                   preferred_element_type=f32)

    def dtg(a, b):
        return jax.lax.dot_general(a, b, (((0,), (0,)), ((), ())),
                                   preferred_element_type=f32)

    def mm(a, b):
        return jnp.dot(a, b, preferred_element_type=f32)

    def norm_t(a):
        n2 = mm(g, a * a)
        return a / jnp.maximum(jnp.sqrt(n2), 1e-12)

    qt = norm_t(dgt(mm(pad, wq_ref[...]), x))
    kt = norm_t(dgt(mm(pad, wk_ref[...]), x))
    vt = dgt(wv_ref[...], x)

    bdt = mm(mm(pad, wn_ref[...]), tile4) * g
    qt = norm_t(dtg(bdt, qt)) * (SCALE * LOG2E) + onec_ref[...]

    bdr = mm(mm(r8_ref[...], roots_ref[...]), tile4) * g
    ssg = jax.nn.sigmoid(mm(bdr, kt) * 3.0)
    u = mm(c1_ref[...], ssg) + o1_ref[...]
    u = mm(b2_ref[...], u) * (mm(c2_ref[...], ssg) + o2_ref[...])
    u = mm(b3_ref[...], u) * (mm(c3_ref[...], ssg) + o3_ref[...])
    u = mm(b4_ref[...], u) * (mm(c4_ref[...], ssg) + o4_ref[...])
    cbe = mm(sel4_ref[...], cb_ref[...])
    cbe = mm(cbe, tile16_ref[...]) * blk16_ref[...]
    kt = kt + mm(cbe, u)

    qtb = qt.astype(bf16)
    ktb = kt.astype(bf16)
    vtb = vt.astype(bf16)
    amask4 = amask_ref[...]
    qmask = qmask_ref[...]
    ones_row = jnp.ones((1, t), bf16)

    def sdot(ks, qs):
        s = jax.lax.dot_general(ks, qs, (((0,), (0,)), ((), ())),
                                preferred_element_type=f32)
        return s.astype(bf16)

    for gq in range(H // 4):
        kg = ktb[gq * 32:(gq + 1) * 32, :]
        qg = qtb[gq * 32:(gq + 1) * 32, :]
        vhs = [jnp.concatenate([vtb[h * DV:(h + 1) * DV, :], ones_row],
                               axis=0) for h in range(gq * 4, gq * 4 + 4)]
        for qb in range(t // BQ):
            lo = qb * BQ
            qq = qg[:, lo:lo + BQ]
            qw = jnp.concatenate([qq, qq, qq, qq], axis=1) * qmask
            p1 = jnp.exp2(sdot(kg[:, lo:lo + BQ], qw) + amask4)
            p0 = jnp.exp2(sdot(kg[:, :lo], qw)) if qb else None
            for a in range(4):
                h = gq * 4 + a
                cs = a * BQ
                oa = mm(vhs[a][:, lo:lo + BQ], p1[:, cs:cs + BQ])
                if qb:
                    oa = oa + mm(vhs[a][:, :lo], p0[:, cs:cs + BQ])
                ot_ref[h * DV:(h + 1) * DV, lo:lo + BQ] = oa[:DV, :]
                l_ref[h:h + 1, lo:lo + BQ] = oa[DV:DV + 1, :]
    lbig = mm(exp_ref[...], 1.0 / l_ref[...])
    y_ref[...] = jax.lax.dot_general(ot_ref[...] * lbig, wout_ref[...],
                                     (((0,), (1,)), ((), ())),
                                     preferred_element_type=f32)


def kernel(x, Wq, Wk, Wv, Wout, W_nudge, chamber_bonus, simple_roots):
    b, t, d_model = x.shape
    c1, o1, b2, c2, o2, b3, c3, o3, b4, c4, o4 = (jnp.asarray(m)
                                                  for m in _CH)
    y = pl.pallas_call(
        _fused_kernel,
        out_shape=jax.ShapeDtypeStruct((t, d_model), jnp.float32),
        scratch_shapes=[pltpu.VMEM((H * DV, t), jnp.float32),
                        pltpu.VMEM((H, t), jnp.float32)],
    )(x.reshape(t, d_model), Wq, Wk, Wv, W_nudge.reshape(H * DH, DH),
      simple_roots, chamber_bonus, jnp.asarray(_PAD), jnp.asarray(_TILE4),
      jnp.asarray(_R8), jnp.asarray(_SEL4), jnp.asarray(_TILE16),
      jnp.asarray(_BLK16), jnp.asarray(_G96), jnp.asarray(_ONEC), c1, o1,
      b2, c2, o2, b3, c3, o3, b4, c4, o4,
      jnp.asarray(_AMASK4, dtype=jnp.bfloat16),
      jnp.asarray(_QMASK, dtype=jnp.bfloat16), jnp.asarray(_EXPD), Wout)
    return y.reshape(b, t, d_model)


# speedup vs baseline: 1.3493x; 1.0318x over previous
import math

import jax
import jax.numpy as jnp
import numpy as np
from jax.experimental import pallas as pl
from jax.experimental.pallas import tpu as pltpu

H = 12
DH = 4
DV = 16
SCALE = 1.0 / math.sqrt(DH)
LOG2E = 1.4426950408889634
BQ = 512


def _np_chamber_mats():
    c1 = np.zeros((24, 96), np.float32)
    o1 = np.zeros((24, 1), np.float32)
    for h in range(H):
        for j in range(2):
            c1[h * 2 + j, h * 8 + 0] = 2 * j - 1
            o1[h * 2 + j, 0] = 1 - j
    b2 = np.zeros((48, 24), np.float32)
    c2 = np.zeros((48, 96), np.float32)
    o2 = np.zeros((48, 1), np.float32)
    for h in range(H):
        for c in range(4):
            j = (c >> 1) & 1
            b2[h * 4 + c, h * 2 + (c & 1)] = 1.0
            c2[h * 4 + c, h * 8 + 1] = 2 * j - 1
            o2[h * 4 + c, 0] = 1 - j
    b3 = np.zeros((96, 48), np.float32)
    c3 = np.zeros((96, 96), np.float32)
    o3 = np.zeros((96, 1), np.float32)
    for h in range(H):
        for c in range(8):
            j = (c >> 2) & 1
            b3[h * 8 + c, h * 4 + (c & 3)] = 1.0
            c3[h * 8 + c, h * 8 + 2] = 2 * j - 1
            o3[h * 8 + c, 0] = 1 - j
    b4 = np.zeros((192, 96), np.float32)
    c4 = np.zeros((192, 96), np.float32)
    o4 = np.zeros((192, 1), np.float32)
    for h in range(H):
        for c in range(16):
            j = (c >> 3) & 1
            b4[h * 16 + c, h * 8 + (c & 7)] = 1.0
            c4[h * 16 + c, h * 8 + 3] = 2 * j - 1
            o4[h * 16 + c, 0] = 1 - j
    return c1, o1, b2, c2, o2, b3, c3, o3, b4, c4, o4


def _np_amask():
    r = np.arange(BQ)[:, None]
    c = np.arange(BQ)[None, :]
    return np.where(r > c, np.float32(-1e30), np.float32(0.0))


def _np_onec():
    o = np.zeros((96, 1), np.float32)
    for h in range(H):
        o[h * 8 + DH, 0] = LOG2E
    return o


_G96 = np.kron(np.eye(H, dtype=np.float32), np.ones((8, 8), np.float32))
_PAD = np.kron(np.eye(H, dtype=np.float32),
               np.eye(8, DH, dtype=np.float32))
_TILE4 = np.kron(np.ones((1, H), np.float32),
                 np.eye(DH, 8, dtype=np.float32))
_R8 = np.kron(np.ones((H, 1), np.float32),
              np.eye(8, DH, dtype=np.float32))
_SEL4 = np.kron(np.eye(H, dtype=np.float32),
                (np.arange(8) == DH).astype(np.float32)[:, None])
_TILE16 = np.kron(np.ones((1, H), np.float32),
                  np.eye(16, dtype=np.float32))
_BLK16 = np.kron(np.eye(H, dtype=np.float32),
                 np.ones((8, 16), np.float32))
_EXPD = np.kron(np.eye(H, dtype=np.float32),
                np.ones((DV, 1), np.float32))
_ONEC = _np_onec()
_CH = _np_chamber_mats()
_AMASK = _np_amask()


def _fused_kernel(x_ref, wq_ref, wk_ref, wv_ref, wn_ref, roots_ref, cb_ref,
                  pad_ref, tile4_ref, r8_ref, sel4_ref, tile16_ref,
                  blk16_ref, g_ref, onec_ref, c1_ref, o1_ref, b2_ref, c2_ref,
                  o2_ref, b3_ref, c3_ref, o3_ref, b4_ref, c4_ref, o4_ref,
                  amask_ref, exp_ref, wout_ref, y_ref, ot_ref, l_ref):
    f32 = jnp.float32
    bf16 = jnp.bfloat16
    x = x_ref[...]
    g = g_ref[...]
    pad = pad_ref[...]
    tile4 = tile4_ref[...]
    t = x.shape[0]

    def dgt(a, b):
        return jax.lax.dot_general(a, b, (((1,), (1,)), ((), ())),
                                   preferred_element_type=f32)

    def dtg(a, b):
        return jax.lax.dot_general(a, b, (((0,), (0,)), ((), ())),
                                   preferred_element_type=f32)

    def mm(a, b):
        return jnp.dot(a, b, preferred_element_type=f32)

    def norm_t(a):
        n2 = mm(g, a * a)
        return a / jnp.maximum(jnp.sqrt(n2), 1e-12)

    qt = norm_t(dgt(mm(pad, wq_ref[...]), x))
    kt = norm_t(dgt(mm(pad, wk_ref[...]), x))
    vt = dgt(wv_ref[...], x)

    bdt = mm(mm(pad, wn_ref[...]), tile4) * g
    qt = norm_t(dtg(bdt, qt)) * (SCALE * LOG2E) + onec_ref[...]

    bdr = mm(mm(r8_ref[...], roots_ref[...]), tile4) * g
    ssg = jax.nn.sigmoid(mm(bdr, kt) * 3.0)
    u = mm(c1_ref[...], ssg) + o1_ref[...]
    u = mm(b2_ref[...], u) * (mm(c2_ref[...], ssg) + o2_ref[...])
    u = mm(b3_ref[...], u) * (mm(c3_ref[...], ssg) + o3_ref[...])
    u = mm(b4_ref[...], u) * (mm(c4_ref[...], ssg) + o4_ref[...])
    cbe = mm(sel4_ref[...], cb_ref[...])
    cbe = mm(cbe, tile16_ref[...]) * blk16_ref[...]
    kt = kt + mm(cbe, u)

    qtb = qt.astype(bf16)
    ktb = kt.astype(bf16)
    vtb = vt.astype(bf16)
    amask = amask_ref[...]
    ones_row = jnp.ones((1, t), bf16)

    def sdot(ks, qs):
        s = jax.lax.dot_general(ks, qs, (((0,), (0,)), ((), ())),
                                preferred_element_type=f32)
        return s.astype(bf16)

    for h in range(H):
        qh = qtb[h * 8:(h + 1) * 8, :]
        kh = ktb[h * 8:(h + 1) * 8, :]
        vh = jnp.concatenate([vtb[h * DV:(h + 1) * DV, :], ones_row], axis=0)
        for qb in range(t // BQ):
            lo = qb * BQ
            qblk = qh[:, lo:lo + BQ]
            p1 = jnp.exp2(sdot(kh[:, lo:lo + BQ], qblk) + amask)
            oa = mm(vh[:, lo:lo + BQ], p1)
            if qb:
                p0 = jnp.exp2(sdot(kh[:, :lo], qblk))
                oa = oa + mm(vh[:, :lo], p0)
            ot_ref[h * DV:(h + 1) * DV, lo:lo + BQ] = oa[:DV, :]
            l_ref[h:h + 1, lo:lo + BQ] = oa[DV:DV + 1, :]
    lbig = mm(exp_ref[...], 1.0 / l_ref[...])
    y_ref[...] = jax.lax.dot_general(ot_ref[...] * lbig, wout_ref[...],
                                     (((0,), (1,)), ((), ())),
                                     preferred_element_type=f32)


def kernel(x, Wq, Wk, Wv, Wout, W_nudge, chamber_bonus, simple_roots):
    b, t, d_model = x.shape
    c1, o1, b2, c2, o2, b3, c3, o3, b4, c4, o4 = (jnp.asarray(m)
                                                  for m in _CH)
    y = pl.pallas_call(
        _fused_kernel,
        out_shape=jax.ShapeDtypeStruct((t, d_model), jnp.float32),
        scratch_shapes=[pltpu.VMEM((H * DV, t), jnp.float32),
                        pltpu.VMEM((H, t), jnp.float32)],
    )(x.reshape(t, d_model), Wq, Wk, Wv, W_nudge.reshape(H * DH, DH),
      simple_roots, chamber_bonus, jnp.asarray(_PAD), jnp.asarray(_TILE4),
      jnp.asarray(_R8), jnp.asarray(_SEL4), jnp.asarray(_TILE16),
      jnp.asarray(_BLK16), jnp.asarray(_G96), jnp.asarray(_ONEC), c1, o1,
      b2, c2, o2, b3, c3, o3, b4, c4, o4,
      jnp.asarray(_AMASK, dtype=jnp.bfloat16), jnp.asarray(_EXPD), Wout)
    return y.reshape(b, t, d_model)
